# baseline (speedup 1.0000x reference)
import functools

import jax
import jax.numpy as jnp
from jax import lax
from jax.experimental import pallas as pl
from jax.experimental.pallas import tpu as pltpu
from jax.experimental.pallas import tpu_sc as plsc

N_NODES = 10000
NUM_FEATURES = 128
HIDDEN_DIM = 64
NUM_CLASSES = 16

NC = 2
NS = 16
NW = NC * NS
K = 128
CHUNKS = 80
E_PAD = NW * K * CHUNKS
N_PAD = 10240
RPT = N_PAD // NS


def _sc_mesh():
    return plsc.VectorSubcoreMesh(core_axis_name="c", subcore_axis_name="s")


_SC_PARAMS = pltpu.CompilerParams(use_tc_tiling_on_sc=False)


DEG_W = 16


_DEG_DEPTH = 8


def _deg_body(dst_hbm, ones_hbm, z_hbm, out_hbm, dst_v, ones_v, acc, sem):
    cid = lax.axis_index("c")
    sid = lax.axis_index("s")
    wid = sid * NC + cid
    pltpu.sync_copy(dst_hbm.at[wid], dst_v)
    pltpu.sync_copy(ones_hbm, ones_v)
    sl = pl.ds(sid * RPT, RPT)
    pltpu.sync_copy(z_hbm, acc.at[sl])
    plsc.subcore_barrier()

    def step(j, carry):
        pltpu.async_copy(ones_v, acc.at[dst_v.at[j]], sem, add=True)
        return carry

    lax.fori_loop(0, _DEG_DEPTH, step, 0)

    def step2(j, carry):
        pltpu.make_async_copy(
            ones_v, acc.at[dst_v.at[j - _DEG_DEPTH]], sem).wait()
        pltpu.async_copy(ones_v, acc.at[dst_v.at[j]], sem, add=True)
        return carry

    lax.fori_loop(_DEG_DEPTH, CHUNKS, step2, 0)

    def drain(j, carry):
        pltpu.make_async_copy(ones_v, acc.at[dst_v.at[j]], sem).wait()
        return carry

    lax.fori_loop(CHUNKS - _DEG_DEPTH, CHUNKS, drain, 0)
    plsc.subcore_barrier()
    pltpu.sync_copy(acc.at[sl], out_hbm.at[cid, sl])


@jax.jit
def _sc_degree(dst3, ones, zeros1):
    return pl.kernel(
        _deg_body,
        out_type=jax.ShapeDtypeStruct((NC, N_PAD, DEG_W), jnp.float32),
        mesh=_sc_mesh(),
        scratch_types=[
            pltpu.VMEM((CHUNKS, K), jnp.int32),
            pltpu.VMEM((K, DEG_W), jnp.float32),
            pltpu.VMEM_SHARED((N_PAD, DEG_W), jnp.float32),
            pltpu.SemaphoreType.DMA,
        ],
        compiler_params=_SC_PARAMS,
    )(dst3, ones, zeros1)


_NBUF = 4


def _agg_body(h_hbm, src_hbm, dst_hbm, z_hbm, out_hbm,
              src_v, dst_v, rows, acc, gsems, ssems):
    cid = lax.axis_index("c")
    sid = lax.axis_index("s")
    wid = sid * NC + cid
    pltpu.sync_copy(src_hbm.at[wid], src_v)
    pltpu.sync_copy(dst_hbm.at[wid], dst_v)
    sl = pl.ds(sid * RPT, RPT)
    pltpu.sync_copy(z_hbm, acc.at[sl])
    plsc.subcore_barrier()

    for b in range(_NBUF):
        pltpu.async_copy(h_hbm.at[src_v.at[b]], rows.at[b], gsems.at[b])

    def step(t, carry):
        j0 = t * _NBUF
        for b in range(_NBUF):
            j = j0 + b
            pltpu.make_async_copy(
                h_hbm.at[src_v.at[j]], rows.at[b], gsems.at[b]).wait()
            pltpu.async_copy(
                rows.at[b], acc.at[dst_v.at[j]], ssems.at[b], add=True)
        for b in range(_NBUF):
            j = j0 + b
            pltpu.make_async_copy(
                rows.at[b], acc.at[dst_v.at[j]], ssems.at[b]).wait()
            pltpu.async_copy(
                h_hbm.at[src_v.at[j + _NBUF]], rows.at[b], gsems.at[b])
        return carry

    lax.fori_loop(0, CHUNKS // _NBUF - 1, step, 0)
    j0 = CHUNKS - _NBUF
    for b in range(_NBUF):
        j = j0 + b
        pltpu.make_async_copy(
            h_hbm.at[src_v.at[j]], rows.at[b], gsems.at[b]).wait()
        pltpu.async_copy(
            rows.at[b], acc.at[dst_v.at[j]], ssems.at[b], add=True)
    for b in range(_NBUF):
        j = j0 + b
        pltpu.make_async_copy(
            rows.at[b], acc.at[dst_v.at[j]], ssems.at[b]).wait()
    plsc.subcore_barrier()
    pltpu.sync_copy(acc.at[sl], out_hbm.at[cid, sl])


@functools.partial(jax.jit, static_argnames=("d",))
def _sc_aggregate(h, src3, dst3, zeros, d):
    return pl.kernel(
        _agg_body,
        out_type=jax.ShapeDtypeStruct((NC, N_PAD, d), jnp.float32),
        mesh=_sc_mesh(),
        scratch_types=[
            pltpu.VMEM((CHUNKS, K), jnp.int32),
            pltpu.VMEM((CHUNKS, K), jnp.int32),
            pltpu.VMEM((_NBUF, K, d), jnp.float32),
            pltpu.VMEM_SHARED((N_PAD, d), jnp.float32),
            pltpu.SemaphoreType.DMA((_NBUF,)),
            pltpu.SemaphoreType.DMA((_NBUF,)),
        ],
        compiler_params=_SC_PARAMS,
    )(h, src3, dst3, zeros)


_BT = 512


def _l1_body(x_ref, w_ref, degp_ref, h_ref, dinv_ref):
    deg = degp_ref[0, :, 0:1] + degp_ref[1, :, 0:1] + 1.0
    dinv = lax.rsqrt(deg)
    h_ref[...] = jnp.dot(x_ref[...], w_ref[...],
                         preferred_element_type=jnp.float32) * dinv
    dinv_ref[...] = dinv


def _tc_layer1(x_pad, w1, degp):
    grid = (N_PAD // _BT,)
    return pl.pallas_call(
        _l1_body,
        grid=grid,
        in_specs=[
            pl.BlockSpec((_BT, NUM_FEATURES), lambda i: (i, 0)),
            pl.BlockSpec((NUM_FEATURES, HIDDEN_DIM), lambda i: (0, 0)),
            pl.BlockSpec((NC, _BT, DEG_W), lambda i: (0, i, 0)),
        ],
        out_specs=[
            pl.BlockSpec((_BT, HIDDEN_DIM), lambda i: (i, 0)),
            pl.BlockSpec((_BT, 1), lambda i: (i, 0)),
        ],
        out_shape=[
            jax.ShapeDtypeStruct((N_PAD, HIDDEN_DIM), jnp.float32),
            jax.ShapeDtypeStruct((N_PAD, 1), jnp.float32),
        ],
    )(x_pad, w1, degp)


def _mid_body(part_ref, h1s_ref, dinv_ref, w2_ref, b1_ref, h2s_ref):
    s1 = part_ref[0] + part_ref[1] + h1s_ref[...]
    dinv = dinv_ref[...]
    g1 = jnp.maximum(s1 * dinv + b1_ref[...], 0.0)
    h2s_ref[...] = jnp.dot(g1, w2_ref[...],
                           preferred_element_type=jnp.float32) * dinv


def _tc_mid(part1, h1s, dinv, w2, b1r):
    grid = (N_PAD // _BT,)
    return pl.pallas_call(
        _mid_body,
        grid=grid,
        in_specs=[
            pl.BlockSpec((NC, _BT, HIDDEN_DIM), lambda i: (0, i, 0)),
            pl.BlockSpec((_BT, HIDDEN_DIM), lambda i: (i, 0)),
            pl.BlockSpec((_BT, 1), lambda i: (i, 0)),
            pl.BlockSpec((HIDDEN_DIM, NUM_CLASSES), lambda i: (0, 0)),
            pl.BlockSpec((1, HIDDEN_DIM), lambda i: (0, 0)),
        ],
        out_specs=pl.BlockSpec((_BT, NUM_CLASSES), lambda i: (i, 0)),
        out_shape=jax.ShapeDtypeStruct((N_PAD, NUM_CLASSES), jnp.float32),
    )(part1, h1s, dinv, w2, b1r)


def _fin_body(part_ref, h2s_ref, dinv_ref, b2_ref, o_ref):
    s2 = part_ref[0] + part_ref[1] + h2s_ref[...]
    o_ref[...] = jax.nn.sigmoid(s2 * dinv_ref[...] + b2_ref[...])


def _tc_final(part2, h2s, dinv, b2r):
    grid = (N_PAD // _BT,)
    return pl.pallas_call(
        _fin_body,
        grid=grid,
        in_specs=[
            pl.BlockSpec((NC, _BT, NUM_CLASSES), lambda i: (0, i, 0)),
            pl.BlockSpec((_BT, NUM_CLASSES), lambda i: (i, 0)),
            pl.BlockSpec((_BT, 1), lambda i: (i, 0)),
            pl.BlockSpec((1, NUM_CLASSES), lambda i: (0, 0)),
        ],
        out_specs=pl.BlockSpec((_BT, NUM_CLASSES), lambda i: (i, 0)),
        out_shape=jax.ShapeDtypeStruct((N_PAD, NUM_CLASSES), jnp.float32),
    )(part2, h2s, dinv, b2r)


def kernel(x, edge_index, W1, b1, W2, b2):
    E = edge_index.shape[1]
    pad = E_PAD - E
    ei = edge_index.astype(jnp.int32)
    src3 = jnp.concatenate(
        [ei[0], jnp.full((pad,), N_NODES, jnp.int32)]).reshape(NW, CHUNKS, K)
    dst3 = jnp.concatenate(
        [ei[1], jnp.full((pad,), N_NODES, jnp.int32)]).reshape(NW, CHUNKS, K)
    x_pad = jnp.pad(x, ((0, N_PAD - N_NODES), (0, 0)))
    ones = jnp.ones((K, DEG_W), jnp.float32)
    z64 = jnp.zeros((RPT, HIDDEN_DIM), jnp.float32)
    z16 = jnp.zeros((RPT, NUM_CLASSES), jnp.float32)

    degp = _sc_degree(dst3, ones, z16)
    h1s, dinv = _tc_layer1(x_pad, W1, degp)
    part1 = _sc_aggregate(h1s, src3, dst3, z64, HIDDEN_DIM)
    h2s = _tc_mid(part1, h1s, dinv, W2, b1.reshape(1, HIDDEN_DIM))
    part2 = _sc_aggregate(h2s, src3, dst3, z16, NUM_CLASSES)
    out = _tc_final(part2, h2s, dinv, b2.reshape(1, NUM_CLASSES))
    return out[:N_NODES]


# speedup vs baseline: 1.6354x; 1.6354x over previous
import functools

import jax
import jax.numpy as jnp
from jax import lax
from jax.experimental import pallas as pl
from jax.experimental.pallas import tpu as pltpu
from jax.experimental.pallas import tpu_sc as plsc

N_NODES = 10000
NUM_FEATURES = 128
HIDDEN_DIM = 64
NUM_CLASSES = 16

NC = 2
NS = 16
NW = NC * NS
K = 128
CHUNKS = 80
E_PAD = NW * K * CHUNKS
N_PAD = 10240
RPT = N_PAD // NS
N_SC = 10016
RSC = N_SC // NS


def _sc_mesh():
    return plsc.VectorSubcoreMesh(core_axis_name="c", subcore_axis_name="s")


_SC_PARAMS = pltpu.CompilerParams(use_tc_tiling_on_sc=False)


DEG_W = 16


_DEG_DEPTH = 8


def _deg_body(sd_hbm, ones_hbm, z_hbm, out_hbm, dst_v, ones_v, acc, sem):
    cid = lax.axis_index("c")
    sid = lax.axis_index("s")
    wid = sid * NC + cid
    pltpu.sync_copy(sd_hbm.at[1, wid], dst_v)
    pltpu.sync_copy(ones_hbm, ones_v)
    sl = pl.ds(sid * RPT, RPT)
    pltpu.sync_copy(z_hbm, acc.at[sl])
    plsc.subcore_barrier()

    def step(j, carry):
        pltpu.async_copy(ones_v, acc.at[dst_v.at[j]], sem, add=True)
        return carry

    lax.fori_loop(0, _DEG_DEPTH, step, 0)

    def step2(j, carry):
        pltpu.make_async_copy(
            ones_v, acc.at[dst_v.at[j - _DEG_DEPTH]], sem).wait()
        pltpu.async_copy(ones_v, acc.at[dst_v.at[j]], sem, add=True)
        return carry

    lax.fori_loop(_DEG_DEPTH, CHUNKS, step2, 0)

    def drain(j, carry):
        pltpu.make_async_copy(ones_v, acc.at[dst_v.at[j]], sem).wait()
        return carry

    lax.fori_loop(CHUNKS - _DEG_DEPTH, CHUNKS, drain, 0)
    plsc.subcore_barrier()
    pltpu.sync_copy(acc.at[sl], out_hbm.at[cid, sl])


@jax.jit
def _sc_degree(sd3, ones, zeros1):
    return pl.kernel(
        _deg_body,
        out_type=jax.ShapeDtypeStruct((NC, N_PAD, DEG_W), jnp.float32),
        mesh=_sc_mesh(),
        scratch_types=[
            pltpu.VMEM((CHUNKS, K), jnp.int32),
            pltpu.VMEM((K, DEG_W), jnp.float32),
            pltpu.VMEM_SHARED((N_PAD, DEG_W), jnp.float32),
            pltpu.SemaphoreType.DMA,
        ],
        compiler_params=_SC_PARAMS,
    )(sd3, ones, zeros1)


_NBUF = 2


def _agg_body(h_hbm, sd_hbm, out_hbm,
              src_v, dst_v, rows, tbl, acc, gsems, ssems):
    d = rows.shape[-1]
    cid = lax.axis_index("c")
    sid = lax.axis_index("s")
    wid = sid * NC + cid
    pltpu.sync_copy(sd_hbm.at[0, wid], src_v)
    pltpu.sync_copy(sd_hbm.at[1, wid], dst_v)
    sl = pl.ds(sid * RSC, RSC)
    pltpu.sync_copy(h_hbm.at[sl], tbl.at[sl])
    zeros16 = jnp.zeros((16,), jnp.float32)

    def zstep(i, carry):
        rows[0, i // (d // 16), pl.ds((i % (d // 16)) * 16, 16)] = zeros16
        return carry

    lax.fori_loop(0, K * d // 16, zstep, 0)
    for q in range(RSC // K):
        pltpu.sync_copy(rows.at[0], acc.at[pl.ds(sid * RSC + q * K, K)])
    rem = RSC % K
    if rem:
        pltpu.sync_copy(rows.at[0, pl.ds(0, rem)],
                        acc.at[pl.ds(sid * RSC + (RSC // K) * K, rem)])
    plsc.subcore_barrier()

    for b in range(_NBUF):
        pltpu.async_copy(tbl.at[src_v.at[b]], rows.at[b], gsems.at[b])

    def step(t, carry):
        j0 = t * _NBUF
        for b in range(_NBUF):
            j = j0 + b
            pltpu.make_async_copy(
                tbl.at[src_v.at[j]], rows.at[b], gsems.at[b]).wait()
            pltpu.async_copy(
                rows.at[b], acc.at[dst_v.at[j]], ssems.at[b], add=True)
        for b in range(_NBUF):
            j = j0 + b
            pltpu.make_async_copy(
                rows.at[b], acc.at[dst_v.at[j]], ssems.at[b]).wait()
            pltpu.async_copy(
                tbl.at[src_v.at[j + _NBUF]], rows.at[b], gsems.at[b])
        return carry

    lax.fori_loop(0, CHUNKS // _NBUF - 1, step, 0)
    j0 = CHUNKS - _NBUF
    for b in range(_NBUF):
        j = j0 + b
        pltpu.make_async_copy(
            tbl.at[src_v.at[j]], rows.at[b], gsems.at[b]).wait()
        pltpu.async_copy(
            rows.at[b], acc.at[dst_v.at[j]], ssems.at[b], add=True)
    for b in range(_NBUF):
        j = j0 + b
        pltpu.make_async_copy(
            rows.at[b], acc.at[dst_v.at[j]], ssems.at[b]).wait()
    plsc.subcore_barrier()
    pltpu.sync_copy(acc.at[sl], out_hbm.at[cid, sl])


@functools.partial(jax.jit, static_argnames=("d",))
def _sc_aggregate(h, sd3, d):
    return pl.kernel(
        _agg_body,
        out_type=jax.ShapeDtypeStruct((NC, N_SC, d), jnp.float32),
        mesh=_sc_mesh(),
        scratch_types=[
            pltpu.VMEM((CHUNKS, K), jnp.int32),
            pltpu.VMEM((CHUNKS, K), jnp.int32),
            pltpu.VMEM((_NBUF, K, d), jnp.float32),
            pltpu.VMEM_SHARED((N_SC, d), jnp.float32),
            pltpu.VMEM_SHARED((N_SC, d), jnp.float32),
            pltpu.SemaphoreType.DMA((_NBUF,)),
            pltpu.SemaphoreType.DMA((_NBUF,)),
        ],
        compiler_params=_SC_PARAMS,
    )(h, sd3)


_BT = 512


def _l1_body(x_ref, w_ref, degp_ref, h_ref, dinv_ref):
    deg = degp_ref[0, :, 0:1] + degp_ref[1, :, 0:1] + 1.0
    dinv = lax.rsqrt(deg)
    h_ref[...] = jnp.dot(x_ref[...], w_ref[...],
                         preferred_element_type=jnp.float32) * dinv
    dinv_ref[...] = dinv


def _tc_layer1(x_pad, w1, degp):
    grid = (N_PAD // _BT,)
    return pl.pallas_call(
        _l1_body,
        grid=grid,
        in_specs=[
            pl.BlockSpec((_BT, NUM_FEATURES), lambda i: (i, 0)),
            pl.BlockSpec((NUM_FEATURES, HIDDEN_DIM), lambda i: (0, 0)),
            pl.BlockSpec((NC, _BT, DEG_W), lambda i: (0, i, 0)),
        ],
        out_specs=[
            pl.BlockSpec((_BT, HIDDEN_DIM), lambda i: (i, 0)),
            pl.BlockSpec((_BT, 1), lambda i: (i, 0)),
        ],
        out_shape=[
            jax.ShapeDtypeStruct((N_PAD, HIDDEN_DIM), jnp.float32),
            jax.ShapeDtypeStruct((N_PAD, 1), jnp.float32),
        ],
    )(x_pad, w1, degp)


def _mid_body(part_ref, h1s_ref, dinv_ref, w2_ref, b1_ref, h2s_ref):
    s1 = part_ref[0] + part_ref[1] + h1s_ref[...]
    dinv = dinv_ref[...]
    g1 = jnp.maximum(s1 * dinv + b1_ref[...], 0.0)
    h2s_ref[...] = jnp.dot(g1, w2_ref[...],
                           preferred_element_type=jnp.float32) * dinv


def _tc_mid(part1, h1s, dinv, w2, b1r):
    grid = (N_PAD // _BT,)
    return pl.pallas_call(
        _mid_body,
        grid=grid,
        in_specs=[
            pl.BlockSpec((NC, _BT, HIDDEN_DIM), lambda i: (0, i, 0)),
            pl.BlockSpec((_BT, HIDDEN_DIM), lambda i: (i, 0)),
            pl.BlockSpec((_BT, 1), lambda i: (i, 0)),
            pl.BlockSpec((HIDDEN_DIM, NUM_CLASSES), lambda i: (0, 0)),
            pl.BlockSpec((1, HIDDEN_DIM), lambda i: (0, 0)),
        ],
        out_specs=pl.BlockSpec((_BT, NUM_CLASSES), lambda i: (i, 0)),
        out_shape=jax.ShapeDtypeStruct((N_PAD, NUM_CLASSES), jnp.float32),
    )(part1, h1s, dinv, w2, b1r)


def _fin_body(part_ref, h2s_ref, dinv_ref, b2_ref, o_ref):
    s2 = part_ref[0] + part_ref[1] + h2s_ref[...]
    o_ref[...] = jax.nn.sigmoid(s2 * dinv_ref[...] + b2_ref[...])


def _tc_final(part2, h2s, dinv, b2r):
    grid = (N_PAD // _BT,)
    return pl.pallas_call(
        _fin_body,
        grid=grid,
        in_specs=[
            pl.BlockSpec((NC, _BT, NUM_CLASSES), lambda i: (0, i, 0)),
            pl.BlockSpec((_BT, NUM_CLASSES), lambda i: (i, 0)),
            pl.BlockSpec((_BT, 1), lambda i: (i, 0)),
            pl.BlockSpec((1, NUM_CLASSES), lambda i: (0, 0)),
        ],
        out_specs=pl.BlockSpec((_BT, NUM_CLASSES), lambda i: (i, 0)),
        out_shape=jax.ShapeDtypeStruct((N_PAD, NUM_CLASSES), jnp.float32),
    )(part2, h2s, dinv, b2r)


def kernel(x, edge_index, W1, b1, W2, b2):
    E = edge_index.shape[1]
    pad = E_PAD - E
    ei = edge_index.astype(jnp.int32)
    src3 = jnp.concatenate(
        [ei[0], jnp.full((pad,), N_NODES, jnp.int32)]).reshape(NW, CHUNKS, K)
    dst3 = jnp.concatenate(
        [ei[1], jnp.full((pad,), N_NODES, jnp.int32)]).reshape(NW, CHUNKS, K)
    x_pad = jnp.pad(x, ((0, N_PAD - N_NODES), (0, 0)))
    ones = jnp.ones((K, DEG_W), jnp.float32)
    z16 = jnp.zeros((RPT, NUM_CLASSES), jnp.float32)
    pad_n = ((0, 0), (0, N_PAD - N_SC), (0, 0))

    sd3 = jnp.stack([src3, dst3])
    degp = _sc_degree(sd3, ones, z16)
    h1s, dinv = _tc_layer1(x_pad, W1, degp)
    part1 = jnp.pad(_sc_aggregate(h1s[:N_SC], sd3, HIDDEN_DIM), pad_n)
    h2s = _tc_mid(part1, h1s, dinv, W2, b1.reshape(1, HIDDEN_DIM))
    part2 = jnp.pad(_sc_aggregate(h2s[:N_SC], sd3, NUM_CLASSES), pad_n)
    out = _tc_final(part2, h2s, dinv, b2.reshape(1, NUM_CLASSES))
    return out[:N_NODES]


# speedup vs baseline: 1.9938x; 1.2191x over previous
import functools

import jax
import jax.numpy as jnp
from jax import lax
from jax.experimental import pallas as pl
from jax.experimental.pallas import tpu as pltpu
from jax.experimental.pallas import tpu_sc as plsc

N_NODES = 10000
NUM_FEATURES = 128
HIDDEN_DIM = 64
NUM_CLASSES = 16

NC = 2
NS = 16
NW = NC * NS
K = 128
CHUNKS = 80
E_PAD = NW * K * CHUNKS
N_PAD = 10240
RPT = N_PAD // NS
N_SC = 10016
RSC = N_SC // NS


def _sc_mesh():
    return plsc.VectorSubcoreMesh(core_axis_name="c", subcore_axis_name="s")


_SC_PARAMS = pltpu.CompilerParams(use_tc_tiling_on_sc=False)


DEG_W = 16


_DEG_DEPTH = 8


def _deg_body(sd_hbm, ones_hbm, z_hbm, out_hbm, dst_v, ones_v, acc, sem):
    cid = lax.axis_index("c")
    sid = lax.axis_index("s")
    wid = sid * NC + cid
    pltpu.sync_copy(sd_hbm.at[1, wid], dst_v)
    pltpu.sync_copy(ones_hbm, ones_v)
    sl = pl.ds(sid * RSC, RSC)
    pltpu.sync_copy(z_hbm, acc.at[sl])
    plsc.subcore_barrier()

    def step(j, carry):
        pltpu.async_copy(ones_v, acc.at[dst_v.at[j]], sem, add=True)
        return carry

    lax.fori_loop(0, _DEG_DEPTH, step, 0)

    def step2(j, carry):
        pltpu.make_async_copy(
            ones_v, acc.at[dst_v.at[j - _DEG_DEPTH]], sem).wait()
        pltpu.async_copy(ones_v, acc.at[dst_v.at[j]], sem, add=True)
        return carry

    lax.fori_loop(_DEG_DEPTH, CHUNKS, step2, 0)

    def drain(j, carry):
        pltpu.make_async_copy(ones_v, acc.at[dst_v.at[j]], sem).wait()
        return carry

    lax.fori_loop(CHUNKS - _DEG_DEPTH, CHUNKS, drain, 0)
    plsc.subcore_barrier()
    pltpu.sync_copy(acc.at[sl], out_hbm.at[cid, sl])


@jax.jit
def _sc_degree(sd3, ones, zeros1):
    return pl.kernel(
        _deg_body,
        out_type=jax.ShapeDtypeStruct((NC, N_SC, DEG_W), jnp.float32),
        mesh=_sc_mesh(),
        scratch_types=[
            pltpu.VMEM((CHUNKS, K), jnp.int32),
            pltpu.VMEM((K, DEG_W), jnp.float32),
            pltpu.VMEM_SHARED((N_SC, DEG_W), jnp.float32),
            pltpu.SemaphoreType.DMA,
        ],
        compiler_params=_SC_PARAMS,
    )(sd3, ones, zeros1)


_NBUF = 2


def _agg_body(h_hbm, sd_hbm, out_hbm,
              src_v, dst_v, rows, tbl, acc, gsems, ssems):
    d = rows.shape[-1]
    cid = lax.axis_index("c")
    sid = lax.axis_index("s")
    wid = sid * NC + cid
    pltpu.sync_copy(sd_hbm.at[0, wid], src_v)
    pltpu.sync_copy(sd_hbm.at[1, wid], dst_v)
    sl = pl.ds(sid * RSC, RSC)
    pltpu.sync_copy(h_hbm.at[sl], tbl.at[sl])
    zeros16 = jnp.zeros((16,), jnp.float32)

    def zstep(i, carry):
        rows[0, i // (d // 16), pl.ds((i % (d // 16)) * 16, 16)] = zeros16
        return carry

    lax.fori_loop(0, K * d // 16, zstep, 0)
    for q in range(RSC // K):
        pltpu.sync_copy(rows.at[0], acc.at[pl.ds(sid * RSC + q * K, K)])
    rem = RSC % K
    if rem:
        pltpu.sync_copy(rows.at[0, pl.ds(0, rem)],
                        acc.at[pl.ds(sid * RSC + (RSC // K) * K, rem)])
    plsc.subcore_barrier()

    for b in range(_NBUF):
        pltpu.async_copy(tbl.at[src_v.at[b]], rows.at[b], gsems.at[b])

    def step(t, carry):
        j0 = t * _NBUF
        for b in range(_NBUF):
            j = j0 + b
            pltpu.make_async_copy(
                tbl.at[src_v.at[j]], rows.at[b], gsems.at[b]).wait()
            pltpu.async_copy(
                rows.at[b], acc.at[dst_v.at[j]], ssems.at[b], add=True)
        for b in range(_NBUF):
            j = j0 + b
            pltpu.make_async_copy(
                rows.at[b], acc.at[dst_v.at[j]], ssems.at[b]).wait()
            pltpu.async_copy(
                tbl.at[src_v.at[j + _NBUF]], rows.at[b], gsems.at[b])
        return carry

    lax.fori_loop(0, CHUNKS // _NBUF - 1, step, 0)
    j0 = CHUNKS - _NBUF
    for b in range(_NBUF):
        j = j0 + b
        pltpu.make_async_copy(
            tbl.at[src_v.at[j]], rows.at[b], gsems.at[b]).wait()
        pltpu.async_copy(
            rows.at[b], acc.at[dst_v.at[j]], ssems.at[b], add=True)
    for b in range(_NBUF):
        j = j0 + b
        pltpu.make_async_copy(
            rows.at[b], acc.at[dst_v.at[j]], ssems.at[b]).wait()
    plsc.subcore_barrier()
    pltpu.sync_copy(acc.at[sl], out_hbm.at[cid, sl])


@functools.partial(jax.jit, static_argnames=("d",))
def _sc_aggregate(h, sd3, d):
    return pl.kernel(
        _agg_body,
        out_type=jax.ShapeDtypeStruct((NC, N_SC, d), jnp.float32),
        mesh=_sc_mesh(),
        scratch_types=[
            pltpu.VMEM((CHUNKS, K), jnp.int32),
            pltpu.VMEM((CHUNKS, K), jnp.int32),
            pltpu.VMEM((_NBUF, K, d), jnp.float32),
            pltpu.VMEM_SHARED((N_SC, d), jnp.float32),
            pltpu.VMEM_SHARED((N_SC, d), jnp.float32),
            pltpu.SemaphoreType.DMA((_NBUF,)),
            pltpu.SemaphoreType.DMA((_NBUF,)),
        ],
        compiler_params=_SC_PARAMS,
    )(h, sd3)


def _mm1_body(x_ref, w_ref, h_ref):
    h_ref[...] = jnp.dot(x_ref[...], w_ref[...],
                         preferred_element_type=jnp.float32)


def _tc_matmul1(x, w1):
    return pl.pallas_call(
        _mm1_body,
        out_shape=jax.ShapeDtypeStruct((N_NODES, HIDDEN_DIM), jnp.float32),
    )(x, w1)


def _scale1_body(h_ref, degp_ref, hs_ref, dinv_ref):
    deg = degp_ref[0, :, 0:1] + degp_ref[1, :, 0:1] + 1.0
    dinv = lax.rsqrt(deg)
    hs_ref[:N_NODES, :] = h_ref[...] * dinv[:N_NODES]
    hs_ref[N_NODES:, :] = jnp.zeros((N_SC - N_NODES, HIDDEN_DIM), jnp.float32)
    dinv_ref[...] = dinv


def _tc_scale1(h1, degp):
    return pl.pallas_call(
        _scale1_body,
        out_shape=[
            jax.ShapeDtypeStruct((N_SC, HIDDEN_DIM), jnp.float32),
            jax.ShapeDtypeStruct((N_SC, 1), jnp.float32),
        ],
    )(h1, degp)


def _mid_body(part_ref, h1s_ref, dinv_ref, w2_ref, b1_ref, h2s_ref):
    s1 = part_ref[0] + part_ref[1] + h1s_ref[...]
    dinv = dinv_ref[...]
    g1 = jnp.maximum(s1 * dinv + b1_ref[...], 0.0)
    h2s_ref[...] = jnp.dot(g1, w2_ref[...],
                           preferred_element_type=jnp.float32) * dinv


def _tc_mid(part1, h1s, dinv, w2, b1r):
    return pl.pallas_call(
        _mid_body,
        out_shape=jax.ShapeDtypeStruct((N_SC, NUM_CLASSES), jnp.float32),
    )(part1, h1s, dinv, w2, b1r)


def _fin_body(part_ref, h2s_ref, dinv_ref, b2_ref, o_ref):
    s2 = part_ref[0] + part_ref[1] + h2s_ref[...]
    o_ref[...] = jax.nn.sigmoid(s2 * dinv_ref[...] + b2_ref[...])


def _tc_final(part2, h2s, dinv, b2r):
    return pl.pallas_call(
        _fin_body,
        out_shape=jax.ShapeDtypeStruct((N_SC, NUM_CLASSES), jnp.float32),
    )(part2, h2s, dinv, b2r)


def kernel(x, edge_index, W1, b1, W2, b2):
    E = edge_index.shape[1]
    ei = edge_index.astype(jnp.int32)
    sd3 = jnp.concatenate(
        [ei, jnp.full((2, E_PAD - E), N_NODES, jnp.int32)],
        axis=1).reshape(2, NW, CHUNKS, K)
    ones = jnp.ones((K, DEG_W), jnp.float32)
    z16 = jnp.zeros((RSC, DEG_W), jnp.float32)

    degp = _sc_degree(sd3, ones, z16)
    h1 = _tc_matmul1(x, W1)
    h1s, dinv = _tc_scale1(h1, degp)
    part1 = _sc_aggregate(h1s, sd3, HIDDEN_DIM)
    h2s = _tc_mid(part1, h1s, dinv, W2, b1.reshape(1, HIDDEN_DIM))
    part2 = _sc_aggregate(h2s, sd3, NUM_CLASSES)
    out = _tc_final(part2, h2s, dinv, b2.reshape(1, NUM_CLASSES))
    return out[:N_NODES]


# speedup vs baseline: 2.0010x; 1.0036x over previous
import functools

import jax
import jax.numpy as jnp
from jax import lax
from jax.experimental import pallas as pl
from jax.experimental.pallas import tpu as pltpu
from jax.experimental.pallas import tpu_sc as plsc

N_NODES = 10000
NUM_FEATURES = 128
HIDDEN_DIM = 64
NUM_CLASSES = 16

NC = 2
NS = 16
NW = NC * NS
K = 128
C0 = 84
C1 = 76
CHUNKS = (C0 + C1) // 2
E_PAD = NS * K * (C0 + C1)
N_PAD = 10240
RPT = N_PAD // NS
N_SC = 10016
RSC = N_SC // NS


def _sc_mesh():
    return plsc.VectorSubcoreMesh(core_axis_name="c", subcore_axis_name="s")


_SC_PARAMS = pltpu.CompilerParams(use_tc_tiling_on_sc=False)


DEG_W = 16


_DEG_DEPTH = 8


def _deg_body(sd_hbm, ones_hbm, z_hbm, out_hbm, dst_v, ones_v, acc, sem):
    cid = lax.axis_index("c")
    sid = lax.axis_index("s")
    wid = sid * NC + cid
    cn = jnp.where(cid == 0, C0, C1)
    pltpu.sync_copy(sd_hbm.at[1, wid], dst_v)
    pltpu.sync_copy(ones_hbm, ones_v)
    sl = pl.ds(sid * RSC, RSC)
    pltpu.sync_copy(z_hbm, acc.at[sl])
    plsc.subcore_barrier()

    def step(j, carry):
        pltpu.async_copy(ones_v, acc.at[dst_v.at[j]], sem, add=True)
        return carry

    lax.fori_loop(0, _DEG_DEPTH, step, 0)

    def step2(j, carry):
        pltpu.make_async_copy(
            ones_v, acc.at[dst_v.at[j - _DEG_DEPTH]], sem).wait()
        pltpu.async_copy(ones_v, acc.at[dst_v.at[j]], sem, add=True)
        return carry

    lax.fori_loop(_DEG_DEPTH, cn, step2, 0)

    def drain(j, carry):
        pltpu.make_async_copy(ones_v, acc.at[dst_v.at[j]], sem).wait()
        return carry

    lax.fori_loop(cn - _DEG_DEPTH, cn, drain, 0)
    plsc.subcore_barrier()
    pltpu.sync_copy(acc.at[sl], out_hbm.at[cid, sl])


@jax.jit
def _sc_degree(sd3, ones, zeros1):
    return pl.kernel(
        _deg_body,
        out_type=jax.ShapeDtypeStruct((NC, N_SC, DEG_W), jnp.float32),
        mesh=_sc_mesh(),
        scratch_types=[
            pltpu.VMEM((C0, K), jnp.int32),
            pltpu.VMEM((K, DEG_W), jnp.float32),
            pltpu.VMEM_SHARED((N_SC, DEG_W), jnp.float32),
            pltpu.SemaphoreType.DMA,
        ],
        compiler_params=_SC_PARAMS,
    )(sd3, ones, zeros1)


_NBUF = 2


def _agg_body(h_hbm, sd_hbm, out_hbm,
              src_v, dst_v, rows, tbl, acc, gsems, ssems):
    d = rows.shape[-1]
    cid = lax.axis_index("c")
    sid = lax.axis_index("s")
    wid = sid * NC + cid
    cn = jnp.where(cid == 0, C0, C1)
    pltpu.sync_copy(sd_hbm.at[0, wid], src_v)
    pltpu.sync_copy(sd_hbm.at[1, wid], dst_v)
    sl = pl.ds(sid * RSC, RSC)
    pltpu.sync_copy(h_hbm.at[sl], tbl.at[sl])
    zeros16 = jnp.zeros((16,), jnp.float32)

    def zstep(i, carry):
        rows[0, i // (d // 16), pl.ds((i % (d // 16)) * 16, 16)] = zeros16
        return carry

    lax.fori_loop(0, K * d // 16, zstep, 0)
    for q in range(RSC // K):
        pltpu.sync_copy(rows.at[0], acc.at[pl.ds(sid * RSC + q * K, K)])
    rem = RSC % K
    if rem:
        pltpu.sync_copy(rows.at[0, pl.ds(0, rem)],
                        acc.at[pl.ds(sid * RSC + (RSC // K) * K, rem)])
    plsc.subcore_barrier()

    for b in range(_NBUF):
        pltpu.async_copy(tbl.at[src_v.at[b]], rows.at[b], gsems.at[b])

    def step(t, carry):
        j0 = t * _NBUF
        for b in range(_NBUF):
            j = j0 + b
            pltpu.make_async_copy(
                tbl.at[src_v.at[j]], rows.at[b], gsems.at[b]).wait()
            pltpu.async_copy(
                rows.at[b], acc.at[dst_v.at[j]], ssems.at[b], add=True)
        for b in range(_NBUF):
            j = j0 + b
            pltpu.make_async_copy(
                rows.at[b], acc.at[dst_v.at[j]], ssems.at[b]).wait()
            pltpu.async_copy(
                tbl.at[src_v.at[j + _NBUF]], rows.at[b], gsems.at[b])
        return carry

    lax.fori_loop(0, cn // _NBUF - 1, step, 0)
    j0 = cn - _NBUF
    for b in range(_NBUF):
        j = j0 + b
        pltpu.make_async_copy(
            tbl.at[src_v.at[j]], rows.at[b], gsems.at[b]).wait()
        pltpu.async_copy(
            rows.at[b], acc.at[dst_v.at[j]], ssems.at[b], add=True)
    for b in range(_NBUF):
        j = j0 + b
        pltpu.make_async_copy(
            rows.at[b], acc.at[dst_v.at[j]], ssems.at[b]).wait()
    plsc.subcore_barrier()
    pltpu.sync_copy(acc.at[sl], out_hbm.at[cid, sl])


@functools.partial(jax.jit, static_argnames=("d",))
def _sc_aggregate(h, sd3, d):
    return pl.kernel(
        _agg_body,
        out_type=jax.ShapeDtypeStruct((NC, N_SC, d), jnp.float32),
        mesh=_sc_mesh(),
        scratch_types=[
            pltpu.VMEM((C0, K), jnp.int32),
            pltpu.VMEM((C0, K), jnp.int32),
            pltpu.VMEM((_NBUF, K, d), jnp.float32),
            pltpu.VMEM_SHARED((N_SC, d), jnp.float32),
            pltpu.VMEM_SHARED((N_SC, d), jnp.float32),
            pltpu.SemaphoreType.DMA((_NBUF,)),
            pltpu.SemaphoreType.DMA((_NBUF,)),
        ],
        compiler_params=_SC_PARAMS,
    )(h, sd3)


def _mm1_body(x_ref, w_ref, h_ref):
    h_ref[...] = jnp.dot(x_ref[...], w_ref[...],
                         preferred_element_type=jnp.float32)


def _tc_matmul1(x, w1):
    return pl.pallas_call(
        _mm1_body,
        out_shape=jax.ShapeDtypeStruct((N_NODES, HIDDEN_DIM), jnp.float32),
    )(x, w1)


def _scale1_body(h_ref, degp_ref, hs_ref, dinv_ref):
    deg = degp_ref[0, :, 0:1] + degp_ref[1, :, 0:1] + 1.0
    dinv = lax.rsqrt(deg)
    hs_ref[:N_NODES, :] = h_ref[...] * dinv[:N_NODES]
    hs_ref[N_NODES:, :] = jnp.zeros((N_SC - N_NODES, HIDDEN_DIM), jnp.float32)
    dinv_ref[...] = dinv


def _tc_scale1(h1, degp):
    return pl.pallas_call(
        _scale1_body,
        out_shape=[
            jax.ShapeDtypeStruct((N_SC, HIDDEN_DIM), jnp.float32),
            jax.ShapeDtypeStruct((N_SC, 1), jnp.float32),
        ],
    )(h1, degp)


def _mid_body(part_ref, h1s_ref, dinv_ref, w2_ref, b1_ref, h2s_ref):
    s1 = part_ref[0] + part_ref[1] + h1s_ref[...]
    dinv = dinv_ref[...]
    g1 = jnp.maximum(s1 * dinv + b1_ref[...], 0.0)
    h2s_ref[...] = jnp.dot(g1, w2_ref[...],
                           preferred_element_type=jnp.float32) * dinv


def _tc_mid(part1, h1s, dinv, w2, b1r):
    return pl.pallas_call(
        _mid_body,
        out_shape=jax.ShapeDtypeStruct((N_SC, NUM_CLASSES), jnp.float32),
    )(part1, h1s, dinv, w2, b1r)


def _fin_body(part_ref, h2s_ref, dinv_ref, b2_ref, o_ref):
    s2 = (part_ref[0, :N_NODES] + part_ref[1, :N_NODES]
          + h2s_ref[:N_NODES])
    o_ref[...] = jax.nn.sigmoid(
        s2 * dinv_ref[:N_NODES] + b2_ref[...])


def _tc_final(part2, h2s, dinv, b2r):
    return pl.pallas_call(
        _fin_body,
        out_shape=jax.ShapeDtypeStruct((N_NODES, NUM_CLASSES), jnp.float32),
    )(part2, h2s, dinv, b2r)


def kernel(x, edge_index, W1, b1, W2, b2):
    E = edge_index.shape[1]
    ei = edge_index.astype(jnp.int32)
    flat = jnp.concatenate(
        [ei, jnp.full((2, E_PAD - E), N_NODES, jnp.int32)], axis=1)
    ev = flat.reshape(2, NS, (C0 + C1) * K)
    e0 = ev[:, :, :C0 * K].reshape(2, NS, 1, C0, K)
    e1 = jnp.pad(
        ev[:, :, C0 * K:].reshape(2, NS, 1, C1, K),
        ((0, 0), (0, 0), (0, 0), (0, C0 - C1), (0, 0)),
        constant_values=N_NODES)
    sd3 = jnp.concatenate([e0, e1], axis=2).reshape(2, NW, C0, K)
    ones = jnp.ones((K, DEG_W), jnp.float32)
    z16 = jnp.zeros((RSC, DEG_W), jnp.float32)

    degp = _sc_degree(sd3, ones, z16)
    h1 = _tc_matmul1(x, W1)
    h1s, dinv = _tc_scale1(h1, degp)
    part1 = _sc_aggregate(h1s, sd3, HIDDEN_DIM)
    h2s = _tc_mid(part1, h1s, dinv, W2, b1.reshape(1, HIDDEN_DIM))
    part2 = _sc_aggregate(h2s, sd3, NUM_CLASSES)
    return _tc_final(part2, h2s, dinv, b2.reshape(1, NUM_CLASSES))


# speedup vs baseline: 2.0184x; 1.0087x over previous
import functools

import jax
import jax.numpy as jnp
from jax import lax
from jax.experimental import pallas as pl
from jax.experimental.pallas import tpu as pltpu
from jax.experimental.pallas import tpu_sc as plsc

N_NODES = 10000
NUM_FEATURES = 128
HIDDEN_DIM = 64
NUM_CLASSES = 16

NC = 2
NS = 16
NW = NC * NS
K = 128
C0 = 84
C1 = 76
CHUNKS = (C0 + C1) // 2
E_PAD = NS * K * (C0 + C1)
N_PAD = 10240
RPT = N_PAD // NS
N_SC = 10016
RSC = N_SC // NS


def _sc_mesh():
    return plsc.VectorSubcoreMesh(core_axis_name="c", subcore_axis_name="s")


_SC_PARAMS = pltpu.CompilerParams(use_tc_tiling_on_sc=False)


DEG_W = 16


_DEG_DEPTH = 16


def _deg_body(sd_hbm, ones_hbm, z_hbm, out_hbm, dst_v, ones_v, acc, sem):
    cid = lax.axis_index("c")
    sid = lax.axis_index("s")
    wid = sid * NC + cid
    cn = jnp.where(cid == 0, C0, C1)

    @pl.when(cid == 0)
    def _():
        pltpu.sync_copy(sd_hbm.at[1, sid, pl.ds(0, C0)], dst_v)

    @pl.when(cid == 1)
    def _():
        pltpu.sync_copy(sd_hbm.at[1, sid, pl.ds(C0, C1)],
                        dst_v.at[pl.ds(0, C1)])

    pltpu.sync_copy(ones_hbm, ones_v)
    sl = pl.ds(sid * RSC, RSC)
    pltpu.sync_copy(z_hbm, acc.at[sl])
    plsc.subcore_barrier()

    def step(j, carry):
        pltpu.async_copy(ones_v, acc.at[dst_v.at[j]], sem, add=True)
        return carry

    lax.fori_loop(0, _DEG_DEPTH, step, 0)

    def step2(j, carry):
        pltpu.make_async_copy(
            ones_v, acc.at[dst_v.at[j - _DEG_DEPTH]], sem).wait()
        pltpu.async_copy(ones_v, acc.at[dst_v.at[j]], sem, add=True)
        return carry

    lax.fori_loop(_DEG_DEPTH, cn, step2, 0)

    def drain(j, carry):
        pltpu.make_async_copy(ones_v, acc.at[dst_v.at[j]], sem).wait()
        return carry

    lax.fori_loop(cn - _DEG_DEPTH, cn, drain, 0)
    plsc.subcore_barrier()
    pltpu.sync_copy(acc.at[sl], out_hbm.at[cid, sl])


@jax.jit
def _sc_degree(sd3, ones, zeros1):
    return pl.kernel(
        _deg_body,
        out_type=jax.ShapeDtypeStruct((NC, N_SC, DEG_W), jnp.float32),
        mesh=_sc_mesh(),
        scratch_types=[
            pltpu.VMEM((C0, K), jnp.int32),
            pltpu.VMEM((K, DEG_W), jnp.float32),
            pltpu.VMEM_SHARED((N_SC, DEG_W), jnp.float32),
            pltpu.SemaphoreType.DMA,
        ],
        compiler_params=_SC_PARAMS,
    )(sd3, ones, zeros1)


_NBUF = 2


def _agg_body(h_hbm, sd_hbm, out_hbm,
              src_v, dst_v, rows, tbl, acc, gsems, ssems):
    d = rows.shape[-1]
    cid = lax.axis_index("c")
    sid = lax.axis_index("s")
    wid = sid * NC + cid
    cn = jnp.where(cid == 0, C0, C1)

    @pl.when(cid == 0)
    def _():
        pltpu.sync_copy(sd_hbm.at[0, sid, pl.ds(0, C0)], src_v)
        pltpu.sync_copy(sd_hbm.at[1, sid, pl.ds(0, C0)], dst_v)

    @pl.when(cid == 1)
    def _():
        pltpu.sync_copy(sd_hbm.at[0, sid, pl.ds(C0, C1)],
                        src_v.at[pl.ds(0, C1)])
        pltpu.sync_copy(sd_hbm.at[1, sid, pl.ds(C0, C1)],
                        dst_v.at[pl.ds(0, C1)])
    sl = pl.ds(sid * RSC, RSC)
    pltpu.sync_copy(h_hbm.at[sl], tbl.at[sl])
    zeros16 = jnp.zeros((16,), jnp.float32)

    def zstep(i, carry):
        rows[0, i // (d // 16), pl.ds((i % (d // 16)) * 16, 16)] = zeros16
        return carry

    lax.fori_loop(0, K * d // 16, zstep, 0)
    for q in range(RSC // K):
        pltpu.sync_copy(rows.at[0], acc.at[pl.ds(sid * RSC + q * K, K)])
    rem = RSC % K
    if rem:
        pltpu.sync_copy(rows.at[0, pl.ds(0, rem)],
                        acc.at[pl.ds(sid * RSC + (RSC // K) * K, rem)])
    plsc.subcore_barrier()

    for b in range(_NBUF):
        pltpu.async_copy(tbl.at[src_v.at[b]], rows.at[b], gsems.at[b])

    def step(t, carry):
        j0 = t * _NBUF
        for b in range(_NBUF):
            j = j0 + b
            pltpu.make_async_copy(
                tbl.at[src_v.at[j]], rows.at[b], gsems.at[b]).wait()
            pltpu.async_copy(
                rows.at[b], acc.at[dst_v.at[j]], ssems.at[b], add=True)
        for b in range(_NBUF):
            j = j0 + b
            pltpu.make_async_copy(
                rows.at[b], acc.at[dst_v.at[j]], ssems.at[b]).wait()
            pltpu.async_copy(
                tbl.at[src_v.at[j + _NBUF]], rows.at[b], gsems.at[b])
        return carry

    lax.fori_loop(0, cn // _NBUF - 1, step, 0)
    j0 = cn - _NBUF
    for b in range(_NBUF):
        j = j0 + b
        pltpu.make_async_copy(
            tbl.at[src_v.at[j]], rows.at[b], gsems.at[b]).wait()
        pltpu.async_copy(
            rows.at[b], acc.at[dst_v.at[j]], ssems.at[b], add=True)
    for b in range(_NBUF):
        j = j0 + b
        pltpu.make_async_copy(
            rows.at[b], acc.at[dst_v.at[j]], ssems.at[b]).wait()
    plsc.subcore_barrier()
    pltpu.sync_copy(acc.at[sl], out_hbm.at[cid, sl])


@functools.partial(jax.jit, static_argnames=("d",))
def _sc_aggregate(h, sd3, d):
    return pl.kernel(
        _agg_body,
        out_type=jax.ShapeDtypeStruct((NC, N_SC, d), jnp.float32),
        mesh=_sc_mesh(),
        scratch_types=[
            pltpu.VMEM((C0, K), jnp.int32),
            pltpu.VMEM((C0, K), jnp.int32),
            pltpu.VMEM((_NBUF, K, d), jnp.float32),
            pltpu.VMEM_SHARED((N_SC, d), jnp.float32),
            pltpu.VMEM_SHARED((N_SC, d), jnp.float32),
            pltpu.SemaphoreType.DMA((_NBUF,)),
            pltpu.SemaphoreType.DMA((_NBUF,)),
        ],
        compiler_params=_SC_PARAMS,
    )(h, sd3)


def _mm1_body(x_ref, w_ref, h_ref):
    h_ref[...] = jnp.dot(x_ref[...], w_ref[...],
                         preferred_element_type=jnp.float32)


def _tc_matmul1(x, w1):
    return pl.pallas_call(
        _mm1_body,
        out_shape=jax.ShapeDtypeStruct((N_NODES, HIDDEN_DIM), jnp.float32),
    )(x, w1)


def _scale1_body(h_ref, degp_ref, hs_ref, dinv_ref):
    deg = degp_ref[0, :, 0:1] + degp_ref[1, :, 0:1] + 1.0
    dinv = lax.rsqrt(deg)
    hs_ref[:N_NODES, :] = h_ref[...] * dinv[:N_NODES]
    hs_ref[N_NODES:, :] = jnp.zeros((N_SC - N_NODES, HIDDEN_DIM), jnp.float32)
    dinv_ref[...] = dinv


def _tc_scale1(h1, degp):
    return pl.pallas_call(
        _scale1_body,
        out_shape=[
            jax.ShapeDtypeStruct((N_SC, HIDDEN_DIM), jnp.float32),
            jax.ShapeDtypeStruct((N_SC, 1), jnp.float32),
        ],
    )(h1, degp)


def _mid_body(part_ref, h1s_ref, dinv_ref, w2_ref, b1_ref, h2s_ref):
    s1 = part_ref[0] + part_ref[1] + h1s_ref[...]
    dinv = dinv_ref[...]
    g1 = jnp.maximum(s1 * dinv + b1_ref[...], 0.0)
    h2s_ref[...] = jnp.dot(g1, w2_ref[...],
                           preferred_element_type=jnp.float32) * dinv


def _tc_mid(part1, h1s, dinv, w2, b1r):
    return pl.pallas_call(
        _mid_body,
        out_shape=jax.ShapeDtypeStruct((N_SC, NUM_CLASSES), jnp.float32),
    )(part1, h1s, dinv, w2, b1r)


def _fin_body(part_ref, h2s_ref, dinv_ref, b2_ref, o_ref):
    s2 = (part_ref[0, :N_NODES] + part_ref[1, :N_NODES]
          + h2s_ref[:N_NODES])
    o_ref[...] = jax.nn.sigmoid(
        s2 * dinv_ref[:N_NODES] + b2_ref[...])


def _tc_final(part2, h2s, dinv, b2r):
    return pl.pallas_call(
        _fin_body,
        out_shape=jax.ShapeDtypeStruct((N_NODES, NUM_CLASSES), jnp.float32),
    )(part2, h2s, dinv, b2r)


def kernel(x, edge_index, W1, b1, W2, b2):
    E = edge_index.shape[1]
    ei = edge_index.astype(jnp.int32)
    sd3 = jnp.concatenate(
        [ei, jnp.full((2, E_PAD - E), N_NODES, jnp.int32)],
        axis=1).reshape(2, NS, C0 + C1, K)
    ones = jnp.ones((K, DEG_W), jnp.float32)
    z16 = jnp.zeros((RSC, DEG_W), jnp.float32)

    degp = _sc_degree(sd3, ones, z16)
    h1 = _tc_matmul1(x, W1)
    h1s, dinv = _tc_scale1(h1, degp)
    part1 = _sc_aggregate(h1s, sd3, HIDDEN_DIM)
    h2s = _tc_mid(part1, h1s, dinv, W2, b1.reshape(1, HIDDEN_DIM))
    part2 = _sc_aggregate(h2s, sd3, NUM_CLASSES)
    return _tc_final(part2, h2s, dinv, b2.reshape(1, NUM_CLASSES))


# speedup vs baseline: 2.0207x; 1.0011x over previous
import functools

import jax
import jax.numpy as jnp
from jax import lax
from jax.experimental import pallas as pl
from jax.experimental.pallas import tpu as pltpu
from jax.experimental.pallas import tpu_sc as plsc

N_NODES = 10000
NUM_FEATURES = 128
HIDDEN_DIM = 64
NUM_CLASSES = 16

NC = 2
NS = 16
NW = NC * NS
K = 128
C0 = 84
C1 = 76
CHUNKS = (C0 + C1) // 2
E_PAD = NS * K * (C0 + C1)
N_PAD = 10240
RPT = N_PAD // NS
N_SC = 10016
RSC = N_SC // NS


def _sc_mesh():
    return plsc.VectorSubcoreMesh(core_axis_name="c", subcore_axis_name="s")


_SC_PARAMS = pltpu.CompilerParams(use_tc_tiling_on_sc=False)


DEG_W = 16


_DEG_DEPTH = 16
TOT_CHUNKS = 320000 // K
_DQ, _DR = divmod(TOT_CHUNKS, NW)


def _deg_body(dst_hbm, out_hbm, dst_v, ones_v, zbuf, acc, sem):
    cid = lax.axis_index("c")
    sid = lax.axis_index("s")
    wid = sid * NC + cid
    base = wid * _DQ + jnp.minimum(wid, _DR)
    cn = _DQ + jnp.where(wid < _DR, 1, 0)

    @pl.when(wid < _DR)
    def _():
        pltpu.sync_copy(dst_hbm.at[pl.ds(base, _DQ + 1)], dst_v)

    @pl.when(wid >= _DR)
    def _():
        pltpu.sync_copy(dst_hbm.at[pl.ds(base, _DQ)],
                        dst_v.at[pl.ds(0, _DQ)])

    ones16 = jnp.ones((16,), jnp.float32)
    zeros16 = jnp.zeros((16,), jnp.float32)

    def fill(i, carry):
        ones_v[i, :] = ones16
        zbuf[i, :] = zeros16
        return carry

    lax.fori_loop(0, K, fill, 0)
    for q in range(RSC // K):
        pltpu.sync_copy(zbuf, acc.at[pl.ds(sid * RSC + q * K, K)])
    rem = RSC % K
    if rem:
        pltpu.sync_copy(zbuf.at[pl.ds(0, rem)],
                        acc.at[pl.ds(sid * RSC + (RSC // K) * K, rem)])
    plsc.subcore_barrier()

    def step(j, carry):
        pltpu.async_copy(ones_v, acc.at[dst_v.at[j]], sem, add=True)
        return carry

    lax.fori_loop(0, _DEG_DEPTH, step, 0)

    def step2(j, carry):
        pltpu.make_async_copy(
            ones_v, acc.at[dst_v.at[j - _DEG_DEPTH]], sem).wait()
        pltpu.async_copy(ones_v, acc.at[dst_v.at[j]], sem, add=True)
        return carry

    lax.fori_loop(_DEG_DEPTH, cn, step2, 0)

    def drain(j, carry):
        pltpu.make_async_copy(ones_v, acc.at[dst_v.at[j]], sem).wait()
        return carry

    lax.fori_loop(cn - _DEG_DEPTH, cn, drain, 0)
    plsc.subcore_barrier()
    sl = pl.ds(sid * RSC, RSC)
    pltpu.sync_copy(acc.at[sl], out_hbm.at[cid, sl])


@jax.jit
def _sc_degree(dstr):
    return pl.kernel(
        _deg_body,
        out_type=jax.ShapeDtypeStruct((NC, N_SC, DEG_W), jnp.float32),
        mesh=_sc_mesh(),
        scratch_types=[
            pltpu.VMEM((_DQ + 1, K), jnp.int32),
            pltpu.VMEM((K, DEG_W), jnp.float32),
            pltpu.VMEM((K, DEG_W), jnp.float32),
            pltpu.VMEM_SHARED((N_SC, DEG_W), jnp.float32),
            pltpu.SemaphoreType.DMA,
        ],
        compiler_params=_SC_PARAMS,
    )(dstr)


_NBUF = 2


def _agg_body(h_hbm, sd_hbm, out_hbm,
              src_v, dst_v, rows, tbl, acc, gsems, ssems):
    d = rows.shape[-1]
    cid = lax.axis_index("c")
    sid = lax.axis_index("s")
    wid = sid * NC + cid
    cn = jnp.where(cid == 0, C0, C1)

    @pl.when(cid == 0)
    def _():
        pltpu.sync_copy(sd_hbm.at[0, sid, pl.ds(0, C0)], src_v)
        pltpu.sync_copy(sd_hbm.at[1, sid, pl.ds(0, C0)], dst_v)

    @pl.when(cid == 1)
    def _():
        pltpu.sync_copy(sd_hbm.at[0, sid, pl.ds(C0, C1)],
                        src_v.at[pl.ds(0, C1)])
        pltpu.sync_copy(sd_hbm.at[1, sid, pl.ds(C0, C1)],
                        dst_v.at[pl.ds(0, C1)])
    sl = pl.ds(sid * RSC, RSC)
    pltpu.sync_copy(h_hbm.at[sl], tbl.at[sl])
    zeros16 = jnp.zeros((16,), jnp.float32)

    def zstep(i, carry):
        rows[0, i // (d // 16), pl.ds((i % (d // 16)) * 16, 16)] = zeros16
        return carry

    lax.fori_loop(0, K * d // 16, zstep, 0)
    for q in range(RSC // K):
        pltpu.sync_copy(rows.at[0], acc.at[pl.ds(sid * RSC + q * K, K)])
    rem = RSC % K
    if rem:
        pltpu.sync_copy(rows.at[0, pl.ds(0, rem)],
                        acc.at[pl.ds(sid * RSC + (RSC // K) * K, rem)])
    plsc.subcore_barrier()

    for b in range(_NBUF):
        pltpu.async_copy(tbl.at[src_v.at[b]], rows.at[b], gsems.at[b])

    def step(t, carry):
        j0 = t * _NBUF
        for b in range(_NBUF):
            j = j0 + b
            pltpu.make_async_copy(
                tbl.at[src_v.at[j]], rows.at[b], gsems.at[b]).wait()
            pltpu.async_copy(
                rows.at[b], acc.at[dst_v.at[j]], ssems.at[b], add=True)
        for b in range(_NBUF):
            j = j0 + b
            pltpu.make_async_copy(
                rows.at[b], acc.at[dst_v.at[j]], ssems.at[b]).wait()
            pltpu.async_copy(
                tbl.at[src_v.at[j + _NBUF]], rows.at[b], gsems.at[b])
        return carry

    lax.fori_loop(0, cn // _NBUF - 1, step, 0)
    j0 = cn - _NBUF
    for b in range(_NBUF):
        j = j0 + b
        pltpu.make_async_copy(
            tbl.at[src_v.at[j]], rows.at[b], gsems.at[b]).wait()
        pltpu.async_copy(
            rows.at[b], acc.at[dst_v.at[j]], ssems.at[b], add=True)
    for b in range(_NBUF):
        j = j0 + b
        pltpu.make_async_copy(
            rows.at[b], acc.at[dst_v.at[j]], ssems.at[b]).wait()
    plsc.subcore_barrier()
    pltpu.sync_copy(acc.at[sl], out_hbm.at[cid, sl])


@functools.partial(jax.jit, static_argnames=("d",))
def _sc_aggregate(h, sd3, d):
    return pl.kernel(
        _agg_body,
        out_type=jax.ShapeDtypeStruct((NC, N_SC, d), jnp.float32),
        mesh=_sc_mesh(),
        scratch_types=[
            pltpu.VMEM((C0, K), jnp.int32),
            pltpu.VMEM((C0, K), jnp.int32),
            pltpu.VMEM((_NBUF, K, d), jnp.float32),
            pltpu.VMEM_SHARED((N_SC, d), jnp.float32),
            pltpu.VMEM_SHARED((N_SC, d), jnp.float32),
            pltpu.SemaphoreType.DMA((_NBUF,)),
            pltpu.SemaphoreType.DMA((_NBUF,)),
        ],
        compiler_params=_SC_PARAMS,
    )(h, sd3)


def _mm1_body(x_ref, w_ref, h_ref):
    h_ref[...] = jnp.dot(x_ref[...], w_ref[...],
                         preferred_element_type=jnp.float32)


def _tc_matmul1(x, w1):
    return pl.pallas_call(
        _mm1_body,
        out_shape=jax.ShapeDtypeStruct((N_NODES, HIDDEN_DIM), jnp.float32),
    )(x, w1)


def _scale1_body(h_ref, degp_ref, hs_ref, dinv_ref):
    deg = degp_ref[0, :, 0:1] + degp_ref[1, :, 0:1] + 1.0
    dinv = lax.rsqrt(deg)
    hs_ref[:N_NODES, :] = h_ref[...] * dinv[:N_NODES]
    hs_ref[N_NODES:, :] = jnp.zeros((N_SC - N_NODES, HIDDEN_DIM), jnp.float32)
    dinv_ref[...] = dinv


def _tc_scale1(h1, degp):
    return pl.pallas_call(
        _scale1_body,
        out_shape=[
            jax.ShapeDtypeStruct((N_SC, HIDDEN_DIM), jnp.float32),
            jax.ShapeDtypeStruct((N_SC, 1), jnp.float32),
        ],
    )(h1, degp)


def _mid_body(part_ref, h1s_ref, dinv_ref, w2_ref, b1_ref, h2s_ref):
    s1 = part_ref[0] + part_ref[1] + h1s_ref[...]
    dinv = dinv_ref[...]
    g1 = jnp.maximum(s1 * dinv + b1_ref[...], 0.0)
    h2s_ref[...] = jnp.dot(g1, w2_ref[...],
                           preferred_element_type=jnp.float32) * dinv


def _tc_mid(part1, h1s, dinv, w2, b1r):
    return pl.pallas_call(
        _mid_body,
        out_shape=jax.ShapeDtypeStruct((N_SC, NUM_CLASSES), jnp.float32),
    )(part1, h1s, dinv, w2, b1r)


def _fin_body(part_ref, h2s_ref, dinv_ref, b2_ref, o_ref):
    s2 = (part_ref[0, :N_NODES] + part_ref[1, :N_NODES]
          + h2s_ref[:N_NODES])
    o_ref[...] = jax.nn.sigmoid(
        s2 * dinv_ref[:N_NODES] + b2_ref[...])


def _tc_final(part2, h2s, dinv, b2r):
    return pl.pallas_call(
        _fin_body,
        out_shape=jax.ShapeDtypeStruct((N_NODES, NUM_CLASSES), jnp.float32),
    )(part2, h2s, dinv, b2r)


def kernel(x, edge_index, W1, b1, W2, b2):
    E = edge_index.shape[1]
    ei = edge_index.astype(jnp.int32)
    sd3 = jnp.concatenate(
        [ei, jnp.full((2, E_PAD - E), N_NODES, jnp.int32)],
        axis=1).reshape(2, NS, C0 + C1, K)
    degp = _sc_degree(ei[1].reshape(TOT_CHUNKS, K))
    h1 = _tc_matmul1(x, W1)
    h1s, dinv = _tc_scale1(h1, degp)
    part1 = _sc_aggregate(h1s, sd3, HIDDEN_DIM)
    h2s = _tc_mid(part1, h1s, dinv, W2, b1.reshape(1, HIDDEN_DIM))
    part2 = _sc_aggregate(h2s, sd3, NUM_CLASSES)
    return _tc_final(part2, h2s, dinv, b2.reshape(1, NUM_CLASSES))
